# Initial kernel scaffold; baseline (speedup 1.0000x reference)
import jax
import jax.numpy as jnp
from jax.experimental import pallas as pl


def kernel(x, table, W, b):
    raise NotImplementedError("write your pallas kernel here")


# speedup vs baseline: 8.3165x; 8.3165x over previous
import functools

import jax
import jax.numpy as jnp
from jax import lax
from jax.experimental import pallas as pl
from jax.experimental.pallas import tpu as pltpu
from jax.experimental.pallas import tpu_sc as plsc

VOCAB = 1000000
EMBED_DIM = 16
BATCH = 16384
HIST = 200

NC = 2
NS = 16
NW = NC * NS

B_PER_W = BATCH // NW
ROWS_PER_CHUNK = 16
CHUNKS = B_PER_W // ROWS_PER_CHUNK
IDX_SPLIT = 2
IDX_PER_GATHER = HIST // IDX_SPLIT
GATHERS_PER_CHUNK = ROWS_PER_CHUNK * IDX_SPLIT
CHUNK_ELEMS = ROWS_PER_CHUNK * HIST

_RNE_MAGIC = float(2.0 ** 23)


def _row_sum(gbuf, base):
  zero = jnp.zeros((EMBED_DIM,), jnp.float32)
  unroll = 20
  iters = HIST // unroll

  def body(i, accs):
    a = list(accs)
    off = base + i * unroll
    for u in range(unroll):
      a[u % 4] = a[u % 4] + gbuf[off + u, :]
    return tuple(a)

  a0, a1, a2, a3 = lax.fori_loop(0, iters, body, (zero, zero, zero, zero))
  return (a0 + a1) + (a2 + a3)


def _make_kernel():
  mesh = plsc.VectorSubcoreMesh(core_axis_name="c", subcore_axis_name="s")

  @functools.partial(
      pl.kernel,
      mesh=mesh,
      compiler_params=pltpu.CompilerParams(
          needs_layout_passes=False, use_tc_tiling_on_sc=False),
      out_type=jax.ShapeDtypeStruct((BATCH,), jnp.float32),
      scratch_types=[
          pltpu.VMEM((GATHERS_PER_CHUNK, IDX_PER_GATHER), jnp.int32),
          pltpu.VMEM((CHUNK_ELEMS, EMBED_DIM), jnp.float32),
          pltpu.VMEM((B_PER_W,), jnp.float32),
          pltpu.VMEM((EMBED_DIM,), jnp.float32),
          pltpu.VMEM((EMBED_DIM,), jnp.float32),
          pltpu.VMEM((EMBED_DIM, ROWS_PER_CHUNK), jnp.float32),
          pltpu.SemaphoreType.DMA,
      ],
  )
  def k(x_hbm, table_hbm, w_hbm, b_hbm, out_hbm,
        idx_v, gbuf, out_v, w_v, b_v, macc, sem):
    wid = lax.axis_index("s") * NC + lax.axis_index("c")
    idx_row0 = wid * (B_PER_W * IDX_SPLIT)

    pltpu.sync_copy(w_hbm, w_v)
    pltpu.sync_copy(b_hbm, b_v)
    wv = w_v[...]
    bv = b_v[...]
    lane = lax.iota(jnp.int32, 16)
    inv_hist = jnp.float32(1.0 / HIST)

    def chunk_body(c, carry):
      pltpu.sync_copy(
          x_hbm.at[pl.ds(idx_row0 + c * GATHERS_PER_CHUNK, GATHERS_PER_CHUNK)],
          idx_v)
      copies = []
      for j in range(GATHERS_PER_CHUNK):
        copies.append(
            pltpu.async_copy(
                table_hbm.at[idx_v.at[j]],
                gbuf.at[pl.ds(j * IDX_PER_GATHER, IDX_PER_GATHER)],
                sem))
      for cp in copies:
        cp.wait()
      for r in range(ROWS_PER_CHUNK):
        acc = _row_sum(gbuf, r * HIST)
        plsc.store_scatter(macc, [lane, jnp.full((16,), r, jnp.int32)],
                           acc * wv)
      cols = [macc[d, :] for d in range(EMBED_DIM)]
      while len(cols) > 1:
        cols = [cols[i] + cols[i + 1] for i in range(0, len(cols), 2)]
      z = cols[0]
      t = z * inv_hist + bv
      p = 1.0 / (1.0 + jnp.exp(-t))
      y = p * jnp.float32(10000.0)
      y = (y + _RNE_MAGIC) - _RNE_MAGIC
      out_v[pl.ds(c * ROWS_PER_CHUNK, ROWS_PER_CHUNK)] = y / jnp.float32(10000.0)
      return carry

    lax.fori_loop(0, CHUNKS, chunk_body, 0)
    pltpu.sync_copy(out_v, out_hbm.at[pl.ds(wid * B_PER_W, B_PER_W)])

  return k


_kernel = _make_kernel()


def kernel(x, table, W, b):
  x2 = x.astype(jnp.int32).reshape(BATCH * IDX_SPLIT, IDX_PER_GATHER)
  wv = W.astype(jnp.float32).reshape(EMBED_DIM)
  bv = jnp.broadcast_to(b.astype(jnp.float32), (EMBED_DIM,))
  out = _kernel(x2, table, wv, bv)
  return out.reshape(BATCH, 1)


# speedup vs baseline: 8.6139x; 1.0358x over previous
import functools

import jax
import jax.numpy as jnp
from jax import lax
from jax.experimental import pallas as pl
from jax.experimental.pallas import tpu as pltpu
from jax.experimental.pallas import tpu_sc as plsc

VOCAB = 1000000
EMBED_DIM = 16
BATCH = 16384
HIST = 200

NC = 2
NS = 16
NW = NC * NS

B_PER_W = BATCH // NW
ROWS_PER_CHUNK = 8
CHUNKS = B_PER_W // ROWS_PER_CHUNK
IDX_SPLIT = 2
IDX_PER_GATHER = HIST // IDX_SPLIT
GATHERS_PER_CHUNK = ROWS_PER_CHUNK * IDX_SPLIT
CHUNK_ELEMS = ROWS_PER_CHUNK * HIST

_RNE_MAGIC = float(2.0 ** 23)


def _row_sum(gbuf, base):
  zero = jnp.zeros((EMBED_DIM,), jnp.float32)
  unroll = 20
  iters = HIST // unroll

  def body(i, accs):
    a = list(accs)
    off = base + i * unroll
    for u in range(unroll):
      a[u % 4] = a[u % 4] + gbuf[off + u, :]
    return tuple(a)

  a0, a1, a2, a3 = lax.fori_loop(0, iters, body, (zero, zero, zero, zero))
  return (a0 + a1) + (a2 + a3)


def _make_kernel():
  mesh = plsc.VectorSubcoreMesh(core_axis_name="c", subcore_axis_name="s")

  @functools.partial(
      pl.kernel,
      mesh=mesh,
      compiler_params=pltpu.CompilerParams(
          needs_layout_passes=False, use_tc_tiling_on_sc=False),
      out_type=jax.ShapeDtypeStruct((BATCH,), jnp.float32),
      scratch_types=[
          pltpu.VMEM((GATHERS_PER_CHUNK, IDX_PER_GATHER), jnp.int32),
          pltpu.VMEM((GATHERS_PER_CHUNK, IDX_PER_GATHER), jnp.int32),
          pltpu.VMEM((CHUNK_ELEMS, EMBED_DIM), jnp.float32),
          pltpu.VMEM((CHUNK_ELEMS, EMBED_DIM), jnp.float32),
          pltpu.VMEM((B_PER_W,), jnp.float32),
          pltpu.VMEM((EMBED_DIM,), jnp.float32),
          pltpu.VMEM((EMBED_DIM,), jnp.float32),
          pltpu.VMEM((EMBED_DIM, 2 * ROWS_PER_CHUNK), jnp.float32),
          pltpu.SemaphoreType.DMA,
          pltpu.SemaphoreType.DMA,
      ],
  )
  def k(x_hbm, table_hbm, w_hbm, b_hbm, out_hbm,
        idx0, idx1, gbuf0, gbuf1, out_v, w_v, b_v, macc, sem0, sem1):
    wid = lax.axis_index("s") * NC + lax.axis_index("c")
    idx_row0 = wid * (B_PER_W * IDX_SPLIT)

    pltpu.sync_copy(w_hbm, w_v)
    pltpu.sync_copy(b_hbm, b_v)
    wv = w_v[...]
    bv = b_v[...]
    lane = lax.iota(jnp.int32, 16)
    inv_hist = jnp.float32(1.0 / HIST)

    def fire(c, idx_v, gbuf, sem):
      pltpu.sync_copy(
          x_hbm.at[pl.ds(idx_row0 + c * GATHERS_PER_CHUNK, GATHERS_PER_CHUNK)],
          idx_v)
      for j in range(GATHERS_PER_CHUNK):
        pltpu.async_copy(
            table_hbm.at[idx_v.at[j]],
            gbuf.at[pl.ds(j * IDX_PER_GATHER, IDX_PER_GATHER)],
            sem)

    def drain(gbuf, sem):
      for j in range(GATHERS_PER_CHUNK):
        pltpu.make_async_copy(
            table_hbm.at[idx0.at[0]],
            gbuf.at[pl.ds(j * IDX_PER_GATHER, IDX_PER_GATHER)],
            sem).wait()

    def reduce_into(gbuf, col_base):
      for r in range(ROWS_PER_CHUNK):
        acc = _row_sum(gbuf, r * HIST)
        plsc.store_scatter(macc, [lane, jnp.full((16,), col_base + r,
                                                 jnp.int32)],
                           acc * wv)

    def epilogue(i):
      cols = [macc[d, :] for d in range(EMBED_DIM)]
      while len(cols) > 1:
        cols = [cols[i] + cols[i + 1] for i in range(0, len(cols), 2)]
      z = cols[0]
      t = z * inv_hist + bv
      p = 1.0 / (1.0 + jnp.exp(-t))
      y = p * jnp.float32(10000.0)
      y = (y + _RNE_MAGIC) - _RNE_MAGIC
      out_v[pl.ds(i * 16, 16)] = y / jnp.float32(10000.0)

    fire(0, idx0, gbuf0, sem0)

    def pair_body(i, carry):
      c0 = i * 2
      drain(gbuf0, sem0)
      fire(c0 + 1, idx1, gbuf1, sem1)
      reduce_into(gbuf0, 0)
      drain(gbuf1, sem1)

      @pl.when(c0 + 2 < CHUNKS)
      def _():
        fire(c0 + 2, idx0, gbuf0, sem0)

      reduce_into(gbuf1, ROWS_PER_CHUNK)
      epilogue(i)
      return carry

    lax.fori_loop(0, CHUNKS // 2, pair_body, 0)
    pltpu.sync_copy(out_v, out_hbm.at[pl.ds(wid * B_PER_W, B_PER_W)])

  return k


_kernel = _make_kernel()


def kernel(x, table, W, b):
  x2 = x.astype(jnp.int32).reshape(BATCH * IDX_SPLIT, IDX_PER_GATHER)
  wv = W.astype(jnp.float32).reshape(EMBED_DIM)
  bv = jnp.broadcast_to(b.astype(jnp.float32), (EMBED_DIM,))
  out = _kernel(x2, table, wv, bv)
  return out.reshape(BATCH, 1)


# speedup vs baseline: 9.4493x; 1.0970x over previous
import functools

import jax
import jax.numpy as jnp
from jax import lax
from jax.experimental import pallas as pl
from jax.experimental.pallas import tpu as pltpu
from jax.experimental.pallas import tpu_sc as plsc

VOCAB = 1000000
EMBED_DIM = 16
BATCH = 16384
HIST = 200

NC = 2
NS = 16
NW = NC * NS

HIST_PAD = 208
Q_LEN = VOCAB + 16

B_PER_W = BATCH // NW
ROWS_PER_CHUNK = 8
CHUNKS = B_PER_W // ROWS_PER_CHUNK
IDX_COLS = 128
DESCS_PER_CHUNK = ROWS_PER_CHUNK * HIST_PAD // IDX_COLS
IDX_ROWS_PER_W = B_PER_W * HIST_PAD // IDX_COLS
CHUNK_VALS = ROWS_PER_CHUNK * HIST_PAD
SUPER_CHUNKS = 16
SUPERS = CHUNKS // SUPER_CHUNKS
IDX_ROWS_PER_SUPER = SUPER_CHUNKS * DESCS_PER_CHUNK

_RNE_MAGIC = float(2.0 ** 23)


_QROWS = VOCAB * EMBED_DIM // 128
_QBLK = 8192


def _q_body(t_ref, m_ref, o_ref):
  o_ref[...] = jnp.dot(t_ref[...], m_ref[...],
                       preferred_element_type=jnp.float32)


_q_call = pl.pallas_call(
    _q_body,
    grid=(pl.cdiv(_QROWS, _QBLK),),
    in_specs=[
        pl.BlockSpec((_QBLK, 128), lambda i: (i, 0)),
        pl.BlockSpec((128, 8), lambda i: (0, 0)),
    ],
    out_specs=pl.BlockSpec((_QBLK, 8), lambda i: (i, 0)),
    out_shape=jax.ShapeDtypeStruct((_QROWS, 8), jnp.float32),
)


def _row_sum(gbuf, r):
  base = r * HIST_PAD
  accs = [gbuf[pl.ds(base + 16 * t, 16)] for t in range(4)]
  for t in range(4, 13):
    accs[t % 4] = accs[t % 4] + gbuf[pl.ds(base + 16 * t, 16)]
  return (accs[0] + accs[1]) + (accs[2] + accs[3])


def _make_sc_kernel():
  mesh = plsc.VectorSubcoreMesh(core_axis_name="c", subcore_axis_name="s")

  @functools.partial(
      pl.kernel,
      mesh=mesh,
      compiler_params=pltpu.CompilerParams(
          needs_layout_passes=False, use_tc_tiling_on_sc=False),
      out_type=jax.ShapeDtypeStruct((BATCH,), jnp.float32),
      scratch_types=[
          pltpu.VMEM_SHARED((Q_LEN,), jnp.float32),
          pltpu.VMEM((IDX_ROWS_PER_SUPER, IDX_COLS), jnp.int32),
          pltpu.VMEM((CHUNK_VALS,), jnp.float32),
          pltpu.VMEM((CHUNK_VALS,), jnp.float32),
          pltpu.VMEM((B_PER_W,), jnp.float32),
          pltpu.VMEM((EMBED_DIM,), jnp.float32),
          pltpu.VMEM((EMBED_DIM, 2 * ROWS_PER_CHUNK), jnp.float32),
          pltpu.SemaphoreType.DMA,
          pltpu.SemaphoreType.DMA,
      ],
  )
  def k(x_hbm, q_hbm, b_hbm, out_hbm,
        q_sp, idx_buf, gbuf0, gbuf1, out_v, b_v, macc, sem0, sem1):
    sid = lax.axis_index("s")
    wid = sid * NC + lax.axis_index("c")

    @pl.when(sid == 0)
    def _():
      pltpu.sync_copy(q_hbm, q_sp)

    pltpu.sync_copy(b_hbm, b_v)
    plsc.subcore_barrier()

    bv = b_v[...]
    lane = lax.iota(jnp.int32, 16)
    inv_hist = jnp.float32(1.0 / HIST)

    def fire(c, gbuf, sem):
      for j in range(DESCS_PER_CHUNK):
        pltpu.async_copy(
            q_sp.at[idx_buf.at[c * DESCS_PER_CHUNK + j]],
            gbuf.at[pl.ds(j * IDX_COLS, IDX_COLS)],
            sem)

    def drain(gbuf, sem):
      for j in range(DESCS_PER_CHUNK):
        pltpu.make_async_copy(
            q_sp.at[idx_buf.at[j]],
            gbuf.at[pl.ds(j * IDX_COLS, IDX_COLS)],
            sem).wait()

    def reduce_into(gbuf, col_base):
      for r in range(ROWS_PER_CHUNK):
        acc = _row_sum(gbuf, r)
        plsc.store_scatter(macc, [lane, jnp.full((16,), col_base + r,
                                                 jnp.int32)],
                           acc)

    def epilogue(i):
      cols = [macc[d, :] for d in range(EMBED_DIM)]
      while len(cols) > 1:
        cols = [cols[i] + cols[i + 1] for i in range(0, len(cols), 2)]
      z = cols[0]
      t = z * inv_hist + bv
      p = 1.0 / (1.0 + jnp.exp(-t))
      y = p * jnp.float32(10000.0)
      y = (y + _RNE_MAGIC) - _RNE_MAGIC
      out_v[pl.ds(i * 16, 16)] = y / jnp.float32(10000.0)

    def super_body(s, carry):
      pltpu.sync_copy(
          x_hbm.at[pl.ds(wid * IDX_ROWS_PER_W + s * IDX_ROWS_PER_SUPER,
                         IDX_ROWS_PER_SUPER)],
          idx_buf)
      fire(0, gbuf0, sem0)

      def pair_body(p, carry2):
        c0 = p * 2
        drain(gbuf0, sem0)
        fire(c0 + 1, gbuf1, sem1)
        reduce_into(gbuf0, 0)
        drain(gbuf1, sem1)

        @pl.when(c0 + 2 < SUPER_CHUNKS)
        def _():
          fire(c0 + 2, gbuf0, sem0)

        reduce_into(gbuf1, ROWS_PER_CHUNK)
        epilogue(s * (SUPER_CHUNKS // 2) + p)
        return carry2

      lax.fori_loop(0, SUPER_CHUNKS // 2, pair_body, 0)
      return carry

    lax.fori_loop(0, SUPERS, super_body, 0)
    pltpu.sync_copy(out_v, out_hbm.at[pl.ds(wid * B_PER_W, B_PER_W)])

  return k


_sc_kernel = _make_sc_kernel()


def kernel(x, table, W, b):
  w = W.astype(jnp.float32).reshape(EMBED_DIM)
  m = jnp.kron(jnp.eye(8, dtype=jnp.float32), w.reshape(EMBED_DIM, 1))
  q = _q_call(table.reshape(_QROWS, 128), m).reshape(VOCAB)
  q_pad = jnp.concatenate([q, jnp.zeros((Q_LEN - VOCAB,), jnp.float32)])

  xi = x.astype(jnp.int32)
  x_pad = jnp.concatenate(
      [xi, jnp.full((BATCH, HIST_PAD - HIST), VOCAB, jnp.int32)], axis=1)
  x2 = x_pad.reshape(BATCH * HIST_PAD // IDX_COLS, IDX_COLS)

  bv = jnp.broadcast_to(b.astype(jnp.float32), (EMBED_DIM,))
  out = _sc_kernel(x2, q_pad, bv)
  return out.reshape(BATCH, 1)
